# baseline (speedup 1.0000x reference)
import jax
import jax.numpy as jnp
from jax import lax
from jax.experimental import pallas as pl
from jax.experimental.pallas import tpu as pltpu


def _make_kernel(n_rows, t_tgt):
    def _kernel(src_ref, tgt_ref, w_enc_ref, b_enc_ref, w_out_ref,
                b_out_ref, o_ref, ctx_ref, h_ref):
        j = pl.program_id(0)

        @pl.when(j == 0)
        def _():
            ctx_ref[...] = jnp.tanh(
                jnp.dot(src_ref[...], w_enc_ref[...],
                        preferred_element_type=jnp.float32)
                + b_enc_ref[...]).astype(jnp.bfloat16)

            for i in range(n_rows):
                sl = pl.ds(i * t_tgt, t_tgt)
                ctx = ctx_ref[sl, :]
                e = tgt_ref[sl, :]
                scores = lax.dot_general(
                    e.astype(jnp.bfloat16), ctx, (((1,), (1,)), ((), ())),
                    preferred_element_type=jnp.float32)
                m = jnp.max(scores, axis=-1, keepdims=True)
                p = jnp.exp(scores - m)
                p = p / jnp.sum(p, axis=-1, keepdims=True)
                attn = jnp.dot(p.astype(jnp.bfloat16), ctx,
                               preferred_element_type=jnp.float32)
                h_ref[sl, :] = (e + attn).astype(jnp.bfloat16)

        w = w_out_ref[...].astype(jnp.bfloat16)
        o_ref[...] = (
            jnp.dot(h_ref[...], w, preferred_element_type=jnp.float32)
            + b_out_ref[...])

    return _kernel


def kernel(enc_emb, dec_emb, w_enc, b_enc, w_out, b_out, src, tgt):
    src_emb = enc_emb[src.reshape(-1)]
    tgt_emb = dec_emb[tgt.reshape(-1)]

    B, T_src = src.shape
    _, T_tgt = tgt.shape
    E = enc_emb.shape[1]
    H = dec_emb.shape[1]
    V = w_out.shape[1]

    tile_v = min(2048, V)
    n_vt = V // tile_v

    logits = pl.pallas_call(
        _make_kernel(B, T_tgt),
        out_shape=jax.ShapeDtypeStruct((B * T_tgt, V), jnp.float32),
        grid=(n_vt,),
        in_specs=[
            pl.BlockSpec((B * T_src, E), lambda j: (0, 0)),
            pl.BlockSpec((B * T_tgt, H), lambda j: (0, 0)),
            pl.BlockSpec((E, H), lambda j: (0, 0)),
            pl.BlockSpec((1, H), lambda j: (0, 0)),
            pl.BlockSpec((H, tile_v), lambda j: (0, j)),
            pl.BlockSpec((1, tile_v), lambda j: (0, j)),
        ],
        out_specs=pl.BlockSpec((B * T_tgt, tile_v), lambda j: (0, j)),
        scratch_shapes=[
            pltpu.VMEM((B * T_src, H), jnp.bfloat16),
            pltpu.VMEM((B * T_tgt, H), jnp.bfloat16),
        ],
        compiler_params=pltpu.CompilerParams(
            dimension_semantics=("arbitrary",)),
    )(src_emb, tgt_emb, w_enc, b_enc, w_out, b_out)

    return logits.reshape(B, T_tgt, V)
